# Initial kernel scaffold; baseline (speedup 1.0000x reference)
import jax
import jax.numpy as jnp
from jax.experimental import pallas as pl


def kernel(rna_data, tissue_id, W, b):
    raise NotImplementedError("write your pallas kernel here")


# speedup vs baseline: 8.3227x; 8.3227x over previous
import functools

import jax
import jax.numpy as jnp
from jax import lax
from jax.experimental import pallas as pl
from jax.experimental.pallas import tpu as pltpu
from jax.experimental.pallas import tpu_sc as plsc

_NUM_CODONS = 64
_B, _L = 4096, 2048
_NC, _NS, _LANES = 2, 16, 16
_NW = _NC * _NS
_ROWS_PER_WORKER = _B // _NW
_RC = 16
_NCHUNK = _ROWS_PER_WORKER // _RC
_TBL = 80


def _make_sc_kernel():
    mesh = plsc.VectorSubcoreMesh(core_axis_name="c", subcore_axis_name="s")

    @functools.partial(
        pl.kernel,
        mesh=mesh,
        out_type=jax.ShapeDtypeStruct((_B,), jnp.float32),
        compiler_params=pltpu.CompilerParams(needs_layout_passes=False),
        scratch_types=[
            pltpu.VMEM((_TBL,), jnp.float32),
            pltpu.VMEM((_RC * _L,), jnp.int32),
            pltpu.VMEM((_RC * _L,), jnp.int32),
            pltpu.VMEM((_ROWS_PER_WORKER,), jnp.float32),
            pltpu.SemaphoreType.DMA,
            pltpu.SemaphoreType.DMA,
        ],
    )
    def sc_kernel(rna_hbm, table_hbm, out_hbm, table_v, buf0, buf1, out_v,
                  sem0, sem1):
        wid = lax.axis_index("s") * _NC + lax.axis_index("c")
        base = wid * _ROWS_PER_WORKER

        pltpu.sync_copy(table_hbm, table_v)

        bufs = (buf0, buf1)
        sems = (sem0, sem1)
        idx0 = lax.iota(jnp.int32, _LANES) * _L

        copies = [None, None]
        copies[0] = pltpu.async_copy(
            rna_hbm.at[pl.ds(base * _L, _RC * _L)], buf0, sem0)

        for c in range(_NCHUNK):
            cur = c % 2
            if c + 1 < _NCHUNK:
                copies[1 - cur] = pltpu.async_copy(
                    rna_hbm.at[pl.ds((base + (c + 1) * _RC) * _L, _RC * _L)],
                    bufs[1 - cur], sems[1 - cur])
            copies[cur].wait()
            buf = bufs[cur]

            def body(j, carry, buf=buf):
                acc, cnt, idx = carry
                v = plsc.load_gather(buf, [idx])
                t = plsc.load_gather(table_v, [v])
                return (acc + t,
                        cnt + jnp.minimum(v, 1),
                        idx + 1)

            acc, cnt, _ = lax.fori_loop(
                0, _L, body,
                (jnp.zeros((_LANES,), jnp.float32),
                 jnp.zeros((_LANES,), jnp.int32),
                 idx0),
                unroll=8)

            out_v[pl.ds(c * _RC, _RC)] = acc / cnt.astype(jnp.float32)

        pltpu.sync_copy(out_v, out_hbm.at[pl.ds(base, _ROWS_PER_WORKER)])

    return sc_kernel


_SC_KERNEL = _make_sc_kernel()


def kernel(rna_data, tissue_id, W, b):
    del tissue_id
    table = jnp.zeros((_TBL,), jnp.float32).at[1:_NUM_CODONS + 1].set(W[0] + b[0])
    y = _SC_KERNEL(rna_data.reshape(_B * _L), table)
    return y.reshape(_B, 1)
